# baseline (speedup 1.0000x reference)
import jax
import jax.numpy as jnp
from jax import lax
from jax.experimental import pallas as pl
from jax.experimental.pallas import tpu as pltpu

_VOCAB = 1000000
_EMBED_DIM = 64
_HIDDEN = 128
_OUTPUT = 2
_SEQ_LEN = 16384

_K = 50
_S = _VOCAB // _K
_R = 128


def _bitonic_sort(v):
    lane = lax.broadcasted_iota(jnp.int32, (_R, _R), 1)
    sub = lax.broadcasted_iota(jnp.int32, (_R, _R), 0)
    flat = sub * _R + lane

    def roll(a, sh, axis):
        sh = sh % _R
        if axis == 1:
            return jnp.concatenate([a[:, sh:], a[:, :sh]], axis=1)
        return jnp.concatenate([a[sh:, :], a[:sh, :]], axis=0)

    for klen_log in range(1, 15):
        klen = 1 << klen_log
        asc = (flat & klen) == 0
        for j_log in range(klen_log - 1, -1, -1):
            j = 1 << j_log
            if j < _R:
                lower = (lane & j) == 0
                vp = jnp.where(lower, roll(v, j, 1), roll(v, -j, 1))
            else:
                jj = j // _R
                lower = (sub & jj) == 0
                vp = jnp.where(lower, roll(v, jj, 0), roll(v, -jj, 0))
            lo = jnp.minimum(v, vp)
            hi = jnp.maximum(v, vp)
            v = jnp.where(lower == asc, lo, hi)
    return v


def _body(starts_ref, x_ref, table_ref, vwt_ref, vb_ref, wwt_ref, wb_ref,
          o_ref, acc_ref, sorted_v, sorted_s, sem):
    k = pl.program_id(0)

    @pl.when(k == 0)
    def _init():
        acc_ref[...] = jnp.zeros_like(acc_ref)
        sorted_v[...] = x_ref[...]
        copy = pltpu.make_async_copy(sorted_v, sorted_s, sem)
        copy.start()
        copy.wait()

    start = starts_ref[k]
    end = starts_ref[k + 1]
    base = k * _S

    def hit(p, acc):
        row = jnp.clip(sorted_s[p // _R, p % _R] - base, 0, _S - 1)
        return acc + table_ref[pl.ds(row, 1), :]

    acc_ref[...] = lax.fori_loop(start, end, hit, acc_ref[...])

    @pl.when(k == _K - 1)
    def _finish():
        avg = acc_ref[...] * (1.0 / _SEQ_LEN)
        h = jnp.tanh(
            jnp.dot(avg, vwt_ref[...], precision=lax.Precision.HIGHEST)
            + vb_ref[...]
        )
        o = (
            jnp.dot(h, wwt_ref[...], precision=lax.Precision.HIGHEST)
            + wb_ref[...]
        )
        m = jnp.max(o, axis=1, keepdims=True)
        e = o - m
        lse = jnp.log(jnp.sum(jnp.exp(e), axis=1, keepdims=True))
        o_ref[...] = e - lse


def kernel(x, table, V_w, V_b, W_w, W_b):
    xi = x.astype(jnp.int32)
    slab_bounds = jnp.arange(_K + 1, dtype=jnp.int32) * _S
    starts = jnp.sum(xi[None, :] < slab_bounds[:, None], axis=1).astype(jnp.int32)
    out = pl.pallas_call(
        _body,
        grid_spec=pltpu.PrefetchScalarGridSpec(
            num_scalar_prefetch=1,
            grid=(_K,),
            in_specs=[
                pl.BlockSpec((_R, _R), lambda k, st_s: (0, 0)),
                pl.BlockSpec((_S, _EMBED_DIM), lambda k, st_s: (k, 0)),
                pl.BlockSpec((_EMBED_DIM, _HIDDEN), lambda k, st_s: (0, 0)),
                pl.BlockSpec((1, _HIDDEN), lambda k, st_s: (0, 0)),
                pl.BlockSpec((_HIDDEN, _OUTPUT), lambda k, st_s: (0, 0)),
                pl.BlockSpec((1, _OUTPUT), lambda k, st_s: (0, 0)),
            ],
            out_specs=pl.BlockSpec((1, _OUTPUT), lambda k, st_s: (0, 0)),
            scratch_shapes=[
                pltpu.VMEM((1, _EMBED_DIM), jnp.float32),
                pltpu.VMEM((_R, _R), jnp.int32),
                pltpu.SMEM((_R, _R), jnp.int32),
                pltpu.SemaphoreType.DMA,
            ],
        ),
        out_shape=jax.ShapeDtypeStruct((1, _OUTPUT), jnp.float32),
    )(
        starts,
        xi.reshape(_R, _R),
        table,
        V_w.T,
        V_b.reshape(1, _HIDDEN),
        W_w.T,
        W_b.reshape(1, _OUTPUT),
    )
    return out.reshape(_OUTPUT)


# speedup vs baseline: 1.1713x; 1.1713x over previous
import jax
import jax.numpy as jnp
from jax import lax
from jax.experimental import pallas as pl
from jax.experimental.pallas import tpu as pltpu

_VOCAB = 1000000
_EMBED_DIM = 64
_HIDDEN = 128
_OUTPUT = 2
_SEQ_LEN = 16384

_K = 50
_S = _VOCAB // _K
_R = 128


def _bitonic_sort(v):
    lane = lax.broadcasted_iota(jnp.int32, (_R, _R), 1)
    sub = lax.broadcasted_iota(jnp.int32, (_R, _R), 0)
    flat = sub * _R + lane

    def roll(a, sh, axis):
        sh = sh % _R
        if axis == 1:
            return jnp.concatenate([a[:, sh:], a[:, :sh]], axis=1)
        return jnp.concatenate([a[sh:, :], a[:sh, :]], axis=0)

    for klen_log in range(1, 15):
        klen = 1 << klen_log
        asc = (flat & klen) == 0
        for j_log in range(klen_log - 1, -1, -1):
            j = 1 << j_log
            if j < _R:
                lower = (lane & j) == 0
                vp = jnp.where(lower, roll(v, j, 1), roll(v, -j, 1))
            else:
                jj = j // _R
                lower = (sub & jj) == 0
                vp = jnp.where(lower, roll(v, jj, 0), roll(v, -jj, 0))
            lo = jnp.minimum(v, vp)
            hi = jnp.maximum(v, vp)
            v = jnp.where(lower == asc, lo, hi)
    return v


def _body(starts_ref, x_ref, table_ref, vwt_ref, vb_ref, wwt_ref, wb_ref,
          o_ref, acc_ref, sorted_v, sorted_s, sem):
    k = pl.program_id(0)

    @pl.when(k == 0)
    def _init():
        acc_ref[...] = jnp.zeros_like(acc_ref)
        sorted_v[...] = _bitonic_sort(x_ref[...])
        copy = pltpu.make_async_copy(sorted_v, sorted_s, sem)
        copy.start()
        copy.wait()

    start = starts_ref[k]
    end = starts_ref[k + 1]
    base = k * _S

    def hit(p, acc):
        row = sorted_s[p >> 7, p & 127] - base
        return acc + table_ref[pl.ds(row, 1), :]

    acc_ref[...] = lax.fori_loop(start, end, hit, acc_ref[...])

    @pl.when(k == _K - 1)
    def _finish():
        avg = acc_ref[...] * (1.0 / _SEQ_LEN)
        h = jnp.tanh(
            jnp.dot(avg, vwt_ref[...], precision=lax.Precision.HIGHEST)
            + vb_ref[...]
        )
        o = (
            jnp.dot(h, wwt_ref[...], precision=lax.Precision.HIGHEST)
            + wb_ref[...]
        )
        m = jnp.max(o, axis=1, keepdims=True)
        e = o - m
        lse = jnp.log(jnp.sum(jnp.exp(e), axis=1, keepdims=True))
        o_ref[...] = e - lse


def kernel(x, table, V_w, V_b, W_w, W_b):
    xi = x.astype(jnp.int32)
    slab_bounds = jnp.arange(_K + 1, dtype=jnp.int32) * _S
    starts = jnp.sum(xi[None, :] < slab_bounds[:, None], axis=1).astype(jnp.int32)
    out = pl.pallas_call(
        _body,
        grid_spec=pltpu.PrefetchScalarGridSpec(
            num_scalar_prefetch=1,
            grid=(_K,),
            in_specs=[
                pl.BlockSpec((_R, _R), lambda k, st_s: (0, 0)),
                pl.BlockSpec((_S, _EMBED_DIM), lambda k, st_s: (k, 0)),
                pl.BlockSpec((_EMBED_DIM, _HIDDEN), lambda k, st_s: (0, 0)),
                pl.BlockSpec((1, _HIDDEN), lambda k, st_s: (0, 0)),
                pl.BlockSpec((_HIDDEN, _OUTPUT), lambda k, st_s: (0, 0)),
                pl.BlockSpec((1, _OUTPUT), lambda k, st_s: (0, 0)),
            ],
            out_specs=pl.BlockSpec((1, _OUTPUT), lambda k, st_s: (0, 0)),
            scratch_shapes=[
                pltpu.VMEM((1, _EMBED_DIM), jnp.float32),
                pltpu.VMEM((_R, _R), jnp.int32),
                pltpu.SMEM((_R, _R), jnp.int32),
                pltpu.SemaphoreType.DMA,
            ],
        ),
        out_shape=jax.ShapeDtypeStruct((1, _OUTPUT), jnp.float32),
    )(
        starts,
        xi.reshape(_R, _R),
        table,
        V_w.T,
        V_b.reshape(1, _HIDDEN),
        W_w.T,
        W_b.reshape(1, _OUTPUT),
    )
    return out.reshape(_OUTPUT)


# speedup vs baseline: 1.2276x; 1.0480x over previous
import jax
import jax.numpy as jnp
from jax import lax
from jax.experimental import pallas as pl
from jax.experimental.pallas import tpu as pltpu

_VOCAB = 1000000
_EMBED_DIM = 64
_HIDDEN = 128
_OUTPUT = 2
_SEQ_LEN = 16384

_K = 50
_S = _VOCAB // _K
_R = 128


def _bitonic_sort(v):
    lane = lax.broadcasted_iota(jnp.int32, (_R, _R), 1)
    sub = lax.broadcasted_iota(jnp.int32, (_R, _R), 0)
    flat = sub * _R + lane

    def roll(a, sh, axis):
        sh = sh % _R
        if axis == 1:
            return jnp.concatenate([a[:, sh:], a[:, :sh]], axis=1)
        return jnp.concatenate([a[sh:, :], a[:sh, :]], axis=0)

    for klen_log in range(1, 15):
        klen = 1 << klen_log
        asc = (flat & klen) == 0
        for j_log in range(klen_log - 1, -1, -1):
            j = 1 << j_log
            if j < _R:
                lower = (lane & j) == 0
                vp = jnp.where(lower, roll(v, j, 1), roll(v, -j, 1))
            else:
                jj = j // _R
                lower = (sub & jj) == 0
                vp = jnp.where(lower, roll(v, jj, 0), roll(v, -jj, 0))
            lo = jnp.minimum(v, vp)
            hi = jnp.maximum(v, vp)
            v = jnp.where(lower == asc, lo, hi)
    return v


def _body(starts_ref, x_ref, table_ref, vwt_ref, vb_ref, wwt_ref, wb_ref,
          o_ref, acc_ref, sorted_v, sorted_s, sem):
    k = pl.program_id(0)

    @pl.when(k == 0)
    def _init():
        acc_ref[...] = jnp.zeros_like(acc_ref)
        sorted_v[...] = _bitonic_sort(x_ref[...])
        copy = pltpu.make_async_copy(sorted_v, sorted_s, sem)
        copy.start()
        copy.wait()

    start = starts_ref[k]
    end = starts_ref[k + 1]
    base = k * _S
    nq = (end - start) >> 3

    def hit8(i, accs):
        a = list(accs)
        p = start + (i << 3)
        for t in range(8):
            pt = p + t
            row = sorted_s[pt >> 7, pt & 127] - base
            a[t & 3] = a[t & 3] + table_ref[pl.ds(row, 1), :]
        return tuple(a)

    zero = jnp.zeros_like(acc_ref)
    accs = lax.fori_loop(0, nq, hit8, (acc_ref[...], zero, zero, zero))

    def hit(p, acc):
        row = sorted_s[p >> 7, p & 127] - base
        return acc + table_ref[pl.ds(row, 1), :]

    tail = lax.fori_loop(start + (nq << 3), end, hit, accs[0])
    acc_ref[...] = tail + accs[1] + accs[2] + accs[3]

    @pl.when(k == _K - 1)
    def _finish():
        avg = acc_ref[...] * (1.0 / _SEQ_LEN)
        h = jnp.tanh(
            jnp.dot(avg, vwt_ref[...], precision=lax.Precision.HIGHEST)
            + vb_ref[...]
        )
        o = (
            jnp.dot(h, wwt_ref[...], precision=lax.Precision.HIGHEST)
            + wb_ref[...]
        )
        m = jnp.max(o, axis=1, keepdims=True)
        e = o - m
        lse = jnp.log(jnp.sum(jnp.exp(e), axis=1, keepdims=True))
        o_ref[...] = e - lse


def kernel(x, table, V_w, V_b, W_w, W_b):
    xi = x.astype(jnp.int32)
    slab_bounds = jnp.arange(_K + 1, dtype=jnp.int32) * _S
    starts = jnp.sum(xi[None, :] < slab_bounds[:, None], axis=1).astype(jnp.int32)
    out = pl.pallas_call(
        _body,
        grid_spec=pltpu.PrefetchScalarGridSpec(
            num_scalar_prefetch=1,
            grid=(_K,),
            in_specs=[
                pl.BlockSpec((_R, _R), lambda k, st_s: (0, 0)),
                pl.BlockSpec((_S, _EMBED_DIM), lambda k, st_s: (k, 0)),
                pl.BlockSpec((_EMBED_DIM, _HIDDEN), lambda k, st_s: (0, 0)),
                pl.BlockSpec((1, _HIDDEN), lambda k, st_s: (0, 0)),
                pl.BlockSpec((_HIDDEN, _OUTPUT), lambda k, st_s: (0, 0)),
                pl.BlockSpec((1, _OUTPUT), lambda k, st_s: (0, 0)),
            ],
            out_specs=pl.BlockSpec((1, _OUTPUT), lambda k, st_s: (0, 0)),
            scratch_shapes=[
                pltpu.VMEM((1, _EMBED_DIM), jnp.float32),
                pltpu.VMEM((_R, _R), jnp.int32),
                pltpu.SMEM((_R, _R), jnp.int32),
                pltpu.SemaphoreType.DMA,
            ],
        ),
        out_shape=jax.ShapeDtypeStruct((1, _OUTPUT), jnp.float32),
    )(
        starts,
        xi.reshape(_R, _R),
        table,
        V_w.T,
        V_b.reshape(1, _HIDDEN),
        W_w.T,
        W_b.reshape(1, _OUTPUT),
    )
    return out.reshape(_OUTPUT)


# speedup vs baseline: 1.6519x; 1.3456x over previous
import functools

import jax
import jax.numpy as jnp
from jax import lax
from jax.experimental import pallas as pl
from jax.experimental.pallas import tpu as pltpu
from jax.experimental.pallas import tpu_sc as plsc

_VOCAB = 1000000
_EMBED_DIM = 64
_HIDDEN = 128
_OUTPUT = 2
_SEQ_LEN = 16384

_NC = 2
_NS = 16
_NW = _NC * _NS
_PER_W = _SEQ_LEN // _NW
_CH = 64
_NCHUNK = _PER_W // _CH
_L = 16


def _gather_sum_kernel(
    idx_hbm, table_hbm, out_hbm, idx_v, rows_a, rows_b, acc_v, sem_a, sem_b
):
    c = lax.axis_index("c")
    s = lax.axis_index("s")
    wid = s * _NC + c

    pltpu.sync_copy(idx_hbm.at[wid], idx_v)

    bufs = (rows_a, rows_b)
    sems = (sem_a, sem_b)

    def fire(j, buf, sem):
        handles = []
        for g in range(_CH // _L):
            ivec = idx_v[j, pl.ds(g * _L, _L)]
            for r in range(_L):
                handles.append(
                    pltpu.async_copy(
                        table_hbm.at[ivec[r]], buf.at[g * _L + r], sem
                    )
                )
        return handles

    def accumulate(buf, accs):
        for i in range(_CH):
            accs = tuple(
                accs[k] + buf[i, pl.ds(_L * k, _L)]
                for k in range(_EMBED_DIM // _L)
            )
        return accs

    accs = tuple(jnp.zeros((_L,), jnp.float32) for _ in range(_EMBED_DIM // _L))

    def body(jj, accs):
        j0 = 2 * jj
        h0 = fire(j0, rows_a, sem_a)
        h1 = fire(j0 + 1, rows_b, sem_b)
        for h in h0:
            h.wait()
        accs = accumulate(rows_a, accs)
        for h in h1:
            h.wait()
        return accumulate(rows_b, accs)

    accs = lax.fori_loop(0, _NCHUNK // 2, body, accs)

    for k in range(_EMBED_DIM // _L):
        acc_v[pl.ds(_L * k, _L)] = accs[k]
    pltpu.sync_copy(acc_v, out_hbm.at[wid])


_gather_sum = functools.partial(
    pl.kernel,
    out_type=jax.ShapeDtypeStruct((_NW, _EMBED_DIM), jnp.float32),
    mesh=plsc.VectorSubcoreMesh(core_axis_name="c", subcore_axis_name="s"),
    scratch_types=[
        pltpu.VMEM((_NCHUNK, _CH), jnp.int32),
        pltpu.VMEM((_CH, _EMBED_DIM), jnp.float32),
        pltpu.VMEM((_CH, _EMBED_DIM), jnp.float32),
        pltpu.VMEM((_EMBED_DIM,), jnp.float32),
        pltpu.SemaphoreType.DMA,
        pltpu.SemaphoreType.DMA,
    ],
)(_gather_sum_kernel)


def _mlp_kernel(ps_ref, vwt_ref, vb_ref, wwt_ref, wb_ref, o_ref):
    avg = jnp.sum(ps_ref[...], axis=0, keepdims=True) * (1.0 / _SEQ_LEN)
    h = jnp.tanh(
        jnp.dot(avg, vwt_ref[...], precision=lax.Precision.HIGHEST)
        + vb_ref[...]
    )
    o = jnp.dot(h, wwt_ref[...], precision=lax.Precision.HIGHEST) + wb_ref[...]
    m = jnp.max(o, axis=1, keepdims=True)
    e = o - m
    lse = jnp.log(jnp.sum(jnp.exp(e), axis=1, keepdims=True))
    o_ref[...] = e - lse


def kernel(x, table, V_w, V_b, W_w, W_b):
    idx = x.astype(jnp.int32).reshape(_NW, _NCHUNK, _CH)
    psums = _gather_sum(idx, table)
    out = pl.pallas_call(
        _mlp_kernel,
        out_shape=jax.ShapeDtypeStruct((1, _OUTPUT), jnp.float32),
    )(
        psums,
        V_w.T,
        V_b.reshape(1, _HIDDEN),
        W_w.T,
        W_b.reshape(1, _OUTPUT),
    )
    return out.reshape(_OUTPUT)


# speedup vs baseline: 1.6817x; 1.0180x over previous
import functools

import jax
import jax.numpy as jnp
from jax import lax
from jax.experimental import pallas as pl
from jax.experimental.pallas import tpu as pltpu
from jax.experimental.pallas import tpu_sc as plsc

_VOCAB = 1000000
_EMBED_DIM = 64
_HIDDEN = 128
_OUTPUT = 2
_SEQ_LEN = 16384

_NC = 2
_NS = 16
_NW = _NC * _NS
_PER_W = _SEQ_LEN // _NW
_CH = 64
_NCHUNK = _PER_W // _CH
_L = 16


def _gather_sum_kernel(
    idx_hbm, table_hbm, out_hbm, idx_v, rows_a, acc_v, sem_a
):
    c = lax.axis_index("c")
    s = lax.axis_index("s")
    wid = s * _NC + c

    pltpu.sync_copy(idx_hbm.at[wid], idx_v)

    handles = []
    for j in range(_NCHUNK):
        for g in range(_CH // _L):
            ivec = idx_v[j, pl.ds(g * _L, _L)]
            for r in range(_L):
                handles.append(
                    pltpu.async_copy(
                        table_hbm.at[ivec[r]],
                        rows_a.at[j * _CH + g * _L + r],
                        sem_a,
                    )
                )
    for h in handles:
        h.wait()

    accs = tuple(jnp.zeros((_L,), jnp.float32) for _ in range(_EMBED_DIM // _L))

    def body(i, a):
        return tuple(
            a[k] + rows_a[i, pl.ds(_L * k, _L)]
            for k in range(_EMBED_DIM // _L)
        )

    accs = lax.fori_loop(0, _PER_W, body, accs)

    for k in range(_EMBED_DIM // _L):
        acc_v[pl.ds(_L * k, _L)] = accs[k]
    pltpu.sync_copy(acc_v, out_hbm.at[wid])


_gather_sum = functools.partial(
    pl.kernel,
    out_type=jax.ShapeDtypeStruct((_NW, _EMBED_DIM), jnp.float32),
    mesh=plsc.VectorSubcoreMesh(core_axis_name="c", subcore_axis_name="s"),
    scratch_types=[
        pltpu.VMEM((_NCHUNK, _CH), jnp.int32),
        pltpu.VMEM((_PER_W, _EMBED_DIM), jnp.float32),
        pltpu.VMEM((_EMBED_DIM,), jnp.float32),
        pltpu.SemaphoreType.DMA,
    ],
)(_gather_sum_kernel)


def _mlp_kernel(ps_ref, vwt_ref, vb_ref, wwt_ref, wb_ref, o_ref):
    avg = jnp.sum(ps_ref[...], axis=0, keepdims=True) * (1.0 / _SEQ_LEN)
    h = jnp.tanh(
        jnp.dot(avg, vwt_ref[...], precision=lax.Precision.HIGHEST)
        + vb_ref[...]
    )
    o = jnp.dot(h, wwt_ref[...], precision=lax.Precision.HIGHEST) + wb_ref[...]
    m = jnp.max(o, axis=1, keepdims=True)
    e = o - m
    lse = jnp.log(jnp.sum(jnp.exp(e), axis=1, keepdims=True))
    o_ref[...] = e - lse


def kernel(x, table, V_w, V_b, W_w, W_b):
    idx = x.astype(jnp.int32).reshape(_NW, _NCHUNK, _CH)
    psums = _gather_sum(idx, table)
    out = pl.pallas_call(
        _mlp_kernel,
        out_shape=jax.ShapeDtypeStruct((1, _OUTPUT), jnp.float32),
    )(
        psums,
        V_w.T,
        V_b.reshape(1, _HIDDEN),
        W_w.T,
        W_b.reshape(1, _OUTPUT),
    )
    return out.reshape(_OUTPUT)
